# baseline (speedup 1.0000x reference)
import dataclasses
import functools
import math

import jax
import jax.numpy as jnp
from jax import lax
from jax.experimental import pallas as pl
from jax.experimental.pallas import tpu as pltpu
from jax.experimental.pallas import tpu_sc as plsc

B = 4
S = 2048
D = 768
N = B * S
NC, NS = 2, 16
NW = NC * NS
RPW = N // NW
WPB = S // RPW
G = 32
NCHUNK = RPW // G
LANES = 16
KSCALE = math.sqrt(D)
NOSEP = 2 * S

_SC_CP = pltpu.CompilerParams()
if "needs_layout_passes" in pltpu.CompilerParams.__dataclass_fields__:
    _SC_CP = dataclasses.replace(_SC_CP, needs_layout_passes=False)


@functools.partial(
    pl.kernel,
    out_type=jax.ShapeDtypeStruct((N, D), jnp.float32),
    compiler_params=_SC_CP,
    mesh=plsc.VectorSubcoreMesh(core_axis_name="c", subcore_axis_name="s"),
    scratch_types=[
        pltpu.VMEM((LANES,), jnp.int32),
        pltpu.VMEM((S,), jnp.int32),
        pltpu.VMEM((LANES,), jnp.int32),
        pltpu.VMEM((D,), jnp.float32),
        pltpu.VMEM((D,), jnp.float32),
        pltpu.VMEM((G, D), jnp.float32),
        pltpu.VMEM((G, D), jnp.float32),
        pltpu.VMEM((G, D), jnp.float32),
        pltpu.VMEM((G, D), jnp.float32),
        pltpu.SemaphoreType.DMA,
        pltpu.SemaphoreType.DMA,
        pltpu.SemaphoreType.DMA,
        pltpu.SemaphoreType.DMA,
    ],
)
def _sc_lookup(sep_hbm, xflat_hbm, token_hbm, pos_hbm, seg_hbm, out_hbm,
               sep_v, xrow_v, minv_v, sg0_v, sg1_v, t0, p0, t1, p1,
               st0, sp0, st1, sp1):
    cid = lax.axis_index("c")
    sid = lax.axis_index("s")
    wid = sid * NC + cid
    base = wid * RPW
    bid = wid // WPB
    s0 = (wid % WPB) * RPW

    pltpu.sync_copy(xflat_hbm.at[pl.ds(bid * S, S)], xrow_v)

    slots = ((t0, p0, st0, sp0), (t1, p1, st1, sp1))

    def issue(c, slot):
        tb, pb, st, sp = slot
        pltpu.async_copy(token_hbm.at[xrow_v.at[pl.ds(s0 + c * G, G)]], tb, st)
        pltpu.async_copy(pos_hbm.at[pl.ds(s0 + c * G, G)], pb, sp)

    def wait_slot(slot):
        tb, pb, st, sp = slot
        pltpu.make_async_copy(token_hbm.at[pl.ds(0, G)], tb, st).wait()
        pltpu.make_async_copy(pos_hbm.at[pl.ds(0, G)], pb, sp).wait()

    issue(0, slots[0])
    issue(1, slots[1])

    pltpu.sync_copy(seg_hbm.at[0], sg0_v)
    pltpu.sync_copy(seg_hbm.at[1], sg1_v)

    @pl.loop(0, D // LANES)
    def _(i):
        sl = pl.ds(i * LANES, LANES)
        sg0_v[sl] = sg0_v[sl] * KSCALE
        sg1_v[sl] = sg1_v[sl] * KSCALE

    pltpu.sync_copy(sep_hbm, sep_v)
    lanes = lax.iota(jnp.int32, LANES)
    minv_v[...] = jnp.full((LANES,), NOSEP, jnp.int32)

    @pl.loop(0, S // LANES)
    def _(i):
        vals = xrow_v[pl.ds(i * LANES, LANES)]
        cand = jnp.where(vals == sep_v[...], lanes + i * LANES, NOSEP)
        minv_v[...] = jnp.minimum(minv_v[...], cand)

    p_first = jnp.min(minv_v[...])

    @pl.loop(0, NCHUNK // 2)
    def _(it):
        for k_, slot in enumerate(slots):
            c = 2 * it + k_
            tb, pb = slot[0], slot[1]
            wait_slot(slot)
            jcut = jnp.clip(p_first - (s0 + c * G), 0, G)

            @pl.loop(0, jcut)
            def _(j, tb=tb, pb=pb):
                for c2 in range(D // LANES):
                    sl = pl.ds(c2 * LANES, LANES)
                    tb[j, sl] = (tb[j, sl] + pb[j, sl]) * KSCALE + sg0_v[sl]

            @pl.loop(jcut, G)
            def _(j, tb=tb, pb=pb):
                for c2 in range(D // LANES):
                    sl = pl.ds(c2 * LANES, LANES)
                    tb[j, sl] = (tb[j, sl] + pb[j, sl]) * KSCALE + sg1_v[sl]

            pltpu.sync_copy(tb, out_hbm.at[pl.ds(base + c * G, G)])

            @pl.when(c + 2 < NCHUNK)
            def _(c=c, slot=slot):
                issue(c + 2, slot)


def kernel(x, sep_token, token_table, pos_table, seg_table):
    xflat = x.reshape(N)
    sep_arr = jnp.full((LANES,), sep_token, jnp.int32)
    out = _sc_lookup(sep_arr, xflat, token_table, pos_table, seg_table)
    return out.reshape(B, S, D)


# speedup vs baseline: 1.5851x; 1.5851x over previous
import dataclasses
import functools
import math

import jax
import jax.numpy as jnp
from jax import lax
from jax.experimental import pallas as pl
from jax.experimental.pallas import tpu as pltpu
from jax.experimental.pallas import tpu_sc as plsc

B = 4
S = 2048
D = 768
N = B * S
NC, NS = 2, 16
NW = NC * NS
RPW = N // NW
WPB = S // RPW
G = 32
NCHUNK = RPW // G
LANES = 16
KSCALE = math.sqrt(D)
NOSEP = 2 * S


def _prep_body(pos_ref, seg_ref, out_ref, dseg_ref):
    out_ref[...] = (pos_ref[...] + seg_ref[0][None, :]) * KSCALE
    dseg_ref[...] = jnp.broadcast_to(
        (seg_ref[1] - seg_ref[0])[None, :] * KSCALE, (8, D))


def _make_posk(pos_table, seg_table):
    return pl.pallas_call(
        _prep_body,
        grid=(8,),
        in_specs=[
            pl.BlockSpec((S // 8, D), lambda g: (g, 0)),
            pl.BlockSpec((2, D), lambda g: (0, 0)),
        ],
        out_specs=[
            pl.BlockSpec((S // 8, D), lambda g: (g, 0)),
            pl.BlockSpec((8, D), lambda g: (0, 0)),
        ],
        out_shape=[
            jax.ShapeDtypeStruct((S, D), jnp.float32),
            jax.ShapeDtypeStruct((8, D), jnp.float32),
        ],
    )(pos_table, seg_table)


_SC_CP = pltpu.CompilerParams()
if "needs_layout_passes" in pltpu.CompilerParams.__dataclass_fields__:
    _SC_CP = dataclasses.replace(_SC_CP, needs_layout_passes=False)


@functools.partial(
    pl.kernel,
    out_type=jax.ShapeDtypeStruct((N, D), jnp.float32),
    compiler_params=_SC_CP,
    mesh=plsc.VectorSubcoreMesh(core_axis_name="c", subcore_axis_name="s"),
    scratch_types=[
        pltpu.VMEM((LANES,), jnp.int32),
        pltpu.VMEM((S,), jnp.int32),
        pltpu.VMEM((LANES,), jnp.int32),
        pltpu.VMEM((D,), jnp.float32),
        pltpu.VMEM((G, D), jnp.float32),
        pltpu.VMEM((G, D), jnp.float32),
        pltpu.VMEM((G, D), jnp.float32),
        pltpu.VMEM((G, D), jnp.float32),
        pltpu.SemaphoreType.DMA,
        pltpu.SemaphoreType.DMA,
        pltpu.SemaphoreType.DMA,
        pltpu.SemaphoreType.DMA,
        pltpu.SemaphoreType.DMA,
    ],
)
def _sc_lookup(sep_hbm, xflat_hbm, token_hbm, posk0_hbm, dseg_hbm, out_hbm,
               sep_v, xrow_v, minv_v, dseg_v, t0, p0, t1, p1,
               saux, st0, sp0, st1, sp1):
    cid = lax.axis_index("c")
    sid = lax.axis_index("s")
    wid = sid * NC + cid
    base = wid * RPW
    bid = wid // WPB
    s0 = (wid % WPB) * RPW

    pltpu.sync_copy(sep_hbm, sep_v)
    pltpu.sync_copy(xflat_hbm.at[pl.ds(bid * S, S)], xrow_v)
    cp_ds = pltpu.async_copy(dseg_hbm.at[0], dseg_v, saux)

    slots = ((t0, p0, st0, sp0), (t1, p1, st1, sp1))

    def issue(c, slot):
        tb, pb, st, sp = slot
        pltpu.async_copy(token_hbm.at[xrow_v.at[pl.ds(s0 + c * G, G)]], tb, st)
        pltpu.async_copy(posk0_hbm.at[pl.ds(s0 + c * G, G)], pb, sp)

    def wait_slot(slot):
        tb, pb, st, sp = slot
        pltpu.make_async_copy(token_hbm.at[pl.ds(0, G)], tb, st).wait()
        pltpu.make_async_copy(posk0_hbm.at[pl.ds(0, G)], pb, sp).wait()

    issue(0, slots[0])
    issue(1, slots[1])

    lanes = lax.iota(jnp.int32, LANES)
    minv_v[...] = jnp.full((LANES,), NOSEP, jnp.int32)

    @pl.loop(0, S // LANES)
    def _(i):
        vals = xrow_v[pl.ds(i * LANES, LANES)]
        cand = jnp.where(vals == sep_v[...], lanes + i * LANES, NOSEP)
        minv_v[...] = jnp.minimum(minv_v[...], cand)

    p_first = jnp.min(minv_v[...])
    cp_ds.wait()

    @pl.loop(0, NCHUNK // 2)
    def _(it):
        for k_, slot in enumerate(slots):
            c = 2 * it + k_
            tb, pb = slot[0], slot[1]
            wait_slot(slot)
            jcut = jnp.clip(p_first - (s0 + c * G), 0, G)

            @plsc.parallel_loop(0, jcut)
            def _(j, tb=tb, pb=pb):
                for c2 in range(D // LANES):
                    sl = pl.ds(c2 * LANES, LANES)
                    tb[j, sl] = tb[j, sl] * KSCALE + pb[j, sl]

            @plsc.parallel_loop(jcut, G)
            def _(j, tb=tb, pb=pb):
                for c2 in range(D // LANES):
                    sl = pl.ds(c2 * LANES, LANES)
                    tb[j, sl] = tb[j, sl] * KSCALE + pb[j, sl] + dseg_v[sl]

            pltpu.sync_copy(tb, out_hbm.at[pl.ds(base + c * G, G)])

            @pl.when(c + 2 < NCHUNK)
            def _(c=c, slot=slot):
                issue(c + 2, slot)


def kernel(x, sep_token, token_table, pos_table, seg_table):
    posk0, dsegk = _make_posk(pos_table, seg_table)
    xflat = x.reshape(N)
    sep_vec = jnp.full((LANES,), sep_token, jnp.int32)
    out = _sc_lookup(sep_vec, xflat, token_table, posk0, dsegk)
    return out.reshape(B, S, D)


# speedup vs baseline: 1.7963x; 1.1333x over previous
import dataclasses
import functools
import math

import jax
import jax.numpy as jnp
from jax import lax
from jax.experimental import pallas as pl
from jax.experimental.pallas import tpu as pltpu
from jax.experimental.pallas import tpu_sc as plsc

B = 4
S = 2048
D = 768
N = B * S
NC, NS = 2, 16
NW = NC * NS
SPAN = S // NW
G = 32
HPS = SPAN // G
NCHUNK = B * HPS
LANES = 16
KSCALE = math.sqrt(D)
NOSEP = 2 * S


def _prep_body(sep_ref, x_ref, pos_ref, seg_ref, out_ref, dseg_ref, fs_ref):
    out_ref[...] = (pos_ref[...] + seg_ref[0][None, :]) * KSCALE
    dseg_ref[...] = jnp.broadcast_to(
        (seg_ref[1] - seg_ref[0])[None, :] * KSCALE, (8, D))
    pos_idx = lax.broadcasted_iota(jnp.int32, (B, S), 1)
    cand = jnp.where(x_ref[...] == sep_ref[0], pos_idx, NOSEP)
    first = jnp.min(cand, axis=1, keepdims=True)
    fs_ref[...] = jnp.concatenate(
        [jnp.broadcast_to(first, (B, 128)),
         jnp.full((8 - B, 128), NOSEP, jnp.int32)], axis=0)


def _make_prep(sep_arr, x, pos_table, seg_table):
    return pl.pallas_call(
        _prep_body,
        in_specs=[
            pl.BlockSpec(memory_space=pltpu.SMEM),
            pl.BlockSpec((B, S), lambda: (0, 0)),
            pl.BlockSpec((S, D), lambda: (0, 0)),
            pl.BlockSpec((2, D), lambda: (0, 0)),
        ],
        out_specs=[
            pl.BlockSpec((S, D), lambda: (0, 0)),
            pl.BlockSpec((8, D), lambda: (0, 0)),
            pl.BlockSpec((8, 128), lambda: (0, 0)),
        ],
        out_shape=[
            jax.ShapeDtypeStruct((S, D), jnp.float32),
            jax.ShapeDtypeStruct((8, D), jnp.float32),
            jax.ShapeDtypeStruct((8, 128), jnp.int32),
        ],
    )(sep_arr, x, pos_table, seg_table)


_SC_CP = pltpu.CompilerParams()
if "needs_layout_passes" in pltpu.CompilerParams.__dataclass_fields__:
    _SC_CP = dataclasses.replace(_SC_CP, needs_layout_passes=False)


@functools.partial(
    pl.kernel,
    out_type=jax.ShapeDtypeStruct((N, D), jnp.float32),
    compiler_params=_SC_CP,
    mesh=plsc.VectorSubcoreMesh(core_axis_name="c", subcore_axis_name="s"),
    scratch_types=[
        pltpu.VMEM((B, SPAN), jnp.int32),
        pltpu.VMEM((B, LANES), jnp.int32),
        pltpu.VMEM((D,), jnp.float32),
        pltpu.VMEM((SPAN, D), jnp.float32),
        pltpu.VMEM((G, D), jnp.float32),
        pltpu.VMEM((G, D), jnp.float32),
        pltpu.SemaphoreType.DMA,
        pltpu.SemaphoreType.DMA,
        pltpu.SemaphoreType.DMA,
        pltpu.SemaphoreType.DMA,
    ],
)
def _sc_lookup(fs_hbm, x_hbm, token_hbm, posk0_hbm, dseg_hbm, out_hbm,
               idx_v, fs_v, dseg_v, p_v, t0, t1,
               saux, spn, st0, st1):
    cid = lax.axis_index("c")
    sid = lax.axis_index("s")
    wid = sid * NC + cid
    span0 = wid * SPAN

    cps = [
        pltpu.async_copy(x_hbm.at[b, pl.ds(span0, SPAN)], idx_v.at[b], saux)
        for b in range(B)
    ]
    cp_p = pltpu.async_copy(posk0_hbm.at[pl.ds(span0, SPAN)], p_v, spn)
    cp_ds = pltpu.async_copy(dseg_hbm.at[0], dseg_v, saux)
    cps_fs = [
        pltpu.async_copy(fs_hbm.at[b, pl.ds(0, LANES)], fs_v.at[b], saux)
        for b in range(B)
    ]
    for cp in cps:
        cp.wait()

    slots = ((t0, st0), (t1, st1))

    def issue(c, slot):
        tb, st = slot
        b, h = c // HPS, c % HPS
        pltpu.async_copy(token_hbm.at[idx_v.at[b, pl.ds(h * G, G)]], tb, st)

    def wait_slot(slot):
        tb, st = slot
        pltpu.make_async_copy(token_hbm.at[pl.ds(0, G)], tb, st).wait()

    issue(0, slots[0])
    issue(1, slots[1])

    cp_ds.wait()
    for cp in cps_fs:
        cp.wait()
    p_first = [jnp.min(fs_v[b, pl.ds(0, LANES)]) for b in range(B)]
    cp_p.wait()

    @pl.loop(0, NCHUNK // 2)
    def _(it):
        pf = p_first[B - 1]
        for b in range(B - 1):
            pf = jnp.where(it == b, p_first[b], pf)
        for h, slot in enumerate(slots):
            tb = slot[0]
            wait_slot(slot)
            jcut = jnp.clip(pf - (span0 + h * G), 0, G)

            @plsc.parallel_loop(0, jcut)
            def _(j, tb=tb, h=h):
                for c2 in range(D // LANES):
                    sl = pl.ds(c2 * LANES, LANES)
                    tb[j, sl] = tb[j, sl] * KSCALE + p_v[h * G + j, sl]

            @plsc.parallel_loop(jcut, G)
            def _(j, tb=tb, h=h):
                for c2 in range(D // LANES):
                    sl = pl.ds(c2 * LANES, LANES)
                    tb[j, sl] = (tb[j, sl] * KSCALE
                                 + p_v[h * G + j, sl] + dseg_v[sl])

            pltpu.sync_copy(
                tb, out_hbm.at[pl.ds(it * S + span0 + h * G, G)])

            @pl.when(2 * it + h + 2 < NCHUNK)
            def _(it=it, h=h, slot=slot):
                issue(2 * it + h + 2, slot)


def kernel(x, sep_token, token_table, pos_table, seg_table):
    sep_arr = jnp.asarray(sep_token, jnp.int32).reshape(1)
    posk0, dsegk, firstsep = _make_prep(sep_arr, x, pos_table, seg_table)
    out = _sc_lookup(firstsep, x, token_table, posk0, dsegk)
    return out.reshape(B, S, D)
